# Initial kernel scaffold; baseline (speedup 1.0000x reference)
import jax
import jax.numpy as jnp
from jax.experimental import pallas as pl


def kernel(pixel_values, token_table, position_embedding):
    raise NotImplementedError("write your pallas kernel here")


# speedup vs baseline: 2.7828x; 2.7828x over previous
import functools

import jax
import jax.numpy as jnp
from jax import lax
from jax.experimental import pallas as pl
from jax.experimental.pallas import tpu as pltpu
from jax.experimental.pallas import tpu_sc as plsc

VOCAB = 100000
HIDDEN = 64
NUM_PIXEL = 4096
BATCH = 64

NUM_CORES = 2
NUM_SUBCORES = 16
NUM_WORKERS = NUM_CORES * NUM_SUBCORES
PPW = NUM_PIXEL // NUM_WORKERS
LANES = 16

_mesh = plsc.VectorSubcoreMesh(core_axis_name="c", subcore_axis_name="s")


@functools.partial(
    pl.kernel,
    out_type=jax.ShapeDtypeStruct((BATCH, NUM_PIXEL, HIDDEN), jnp.float32),
    mesh=_mesh,
    scratch_types=[
        pltpu.VMEM((BATCH, PPW), jnp.int32),
        pltpu.VMEM((PPW, HIDDEN), jnp.float32),
        pltpu.VMEM((PPW, HIDDEN), jnp.float32),
        pltpu.SemaphoreType.DMA,
    ],
    compiler_params=pltpu.CompilerParams(use_tc_tiling_on_sc=False),
)
def _emb_kernel(idx_hbm, table_hbm, pos_hbm, out_hbm, idx_v, pos_v, rows_v, sem):
    c = lax.axis_index("c")
    s = lax.axis_index("s")
    w = s * NUM_CORES + c
    base = w * PPW

    pltpu.sync_copy(idx_hbm.at[:, pl.ds(base, PPW)], idx_v)
    pltpu.sync_copy(pos_hbm.at[pl.ds(base, PPW), :], pos_v)

    @pl.loop(0, BATCH)
    def _batch(b):
        pltpu.async_copy(table_hbm.at[idx_v.at[b]], rows_v, sem).wait()

        @pl.loop(0, PPW)
        def _add(i):
            for k in range(HIDDEN // LANES):
                sl = pl.ds(k * LANES, LANES)
                rows_v[i, sl] = rows_v[i, sl] + pos_v[i, sl]

        pltpu.sync_copy(rows_v, out_hbm.at[b, pl.ds(base, PPW), :])


def kernel(pixel_values, token_table, position_embedding):
    idx = pixel_values.astype(jnp.int32)
    pos = position_embedding.reshape(NUM_PIXEL, HIDDEN)
    return _emb_kernel(idx, token_table, pos)


# speedup vs baseline: 2.9831x; 1.0720x over previous
import functools

import jax
import jax.numpy as jnp
from jax import lax
from jax.experimental import pallas as pl
from jax.experimental.pallas import tpu as pltpu
from jax.experimental.pallas import tpu_sc as plsc

VOCAB = 100000
HIDDEN = 64
NUM_PIXEL = 4096
BATCH = 64

NUM_CORES = 2
NUM_SUBCORES = 16
NUM_WORKERS = NUM_CORES * NUM_SUBCORES
PPW = NUM_PIXEL // NUM_WORKERS
LANES = 16
NBUF = 4

_mesh = plsc.VectorSubcoreMesh(core_axis_name="c", subcore_axis_name="s")


@functools.partial(
    pl.kernel,
    out_type=jax.ShapeDtypeStruct((BATCH, NUM_PIXEL, HIDDEN), jnp.float32),
    mesh=_mesh,
    scratch_types=[
        pltpu.VMEM((BATCH, PPW), jnp.int32),
        pltpu.VMEM((PPW, HIDDEN), jnp.float32),
        pltpu.VMEM((NBUF, PPW, HIDDEN), jnp.float32),
        pltpu.VMEM((NBUF, PPW, HIDDEN), jnp.float32),
        [pltpu.SemaphoreType.DMA] * NBUF,
        [pltpu.SemaphoreType.DMA] * NBUF,
    ],
    compiler_params=pltpu.CompilerParams(use_tc_tiling_on_sc=False),
)
def _emb_kernel(idx_hbm, table_hbm, pos_hbm, out_hbm,
                idx_v, pos_v, rows_v, obuf_v, gsems, osems):
    c = lax.axis_index("c")
    s = lax.axis_index("s")
    w = s * NUM_CORES + c
    base = w * PPW

    pltpu.sync_copy(idx_hbm.at[:, pl.ds(base, PPW)], idx_v)
    pltpu.sync_copy(pos_hbm.at[pl.ds(base, PPW), :], pos_v)

    def gather(b, d):
        return pltpu.make_async_copy(
            table_hbm.at[idx_v.at[b]], rows_v.at[d], gsems[d])

    def out_copy(b, d):
        return pltpu.make_async_copy(
            obuf_v.at[d], out_hbm.at[b, pl.ds(base, PPW), :], osems[d])

    for d in range(NBUF):
        gather(d, d).start()

    @pl.loop(0, BATCH, step=NBUF)
    def _group(g):
        for d in range(NBUF):
            b = g + d
            gather(b, d).wait()

            @pl.when(b >= NBUF)
            def _():
                out_copy(b - NBUF, d).wait()

            @pl.loop(0, PPW, unroll=4)
            def _add(i):
                for k in range(HIDDEN // LANES):
                    sl = pl.ds(k * LANES, LANES)
                    obuf_v[d, i, sl] = rows_v[d, i, sl] + pos_v[i, sl]

            out_copy(b, d).start()

            @pl.when(b + NBUF < BATCH)
            def _():
                gather(b + NBUF, d).start()

    for d in range(NBUF):
        out_copy(BATCH - NBUF + d, d).wait()


def kernel(pixel_values, token_table, position_embedding):
    idx = pixel_values.astype(jnp.int32)
    pos = position_embedding.reshape(NUM_PIXEL, HIDDEN)
    return _emb_kernel(idx, token_table, pos)
